# Initial kernel scaffold; baseline (speedup 1.0000x reference)
import functools

import jax
import jax.numpy as jnp
from jax import lax
from jax.experimental import pallas as pl
from jax.experimental.pallas import tpu as pltpu
from jax.experimental.pallas import tpu_sc as plsc

NC = 2
NS = 16
NW = NC * NS

SL = 128
KW = 4
C = KW * SL


def _make_kernel(n_rows, word_dim, pos_dim, out_dim):
    per_w = n_rows // NW
    n_chunks = per_w // C
    mesh = plsc.VectorSubcoreMesh(core_axis_name="c", subcore_axis_name="s",
                                  num_cores=NC, num_subcores=NS)

    @functools.partial(
        pl.kernel,
        out_type=jax.ShapeDtypeStruct((n_rows, out_dim), jnp.float32),
        mesh=mesh,
        scratch_types=[
            pltpu.VMEM((KW, SL), jnp.int32),
            pltpu.VMEM((KW, SL), jnp.int32),
            pltpu.VMEM((KW, SL), jnp.int32),
            pltpu.VMEM((C, word_dim), jnp.float32),
            pltpu.VMEM((C, pos_dim), jnp.float32),
            pltpu.VMEM((C, pos_dim), jnp.float32),
            pltpu.SemaphoreType.DMA,
        ],
    )
    def k(widx, p1idx, p2idx, wtab, p1tab, p2tab, out,
          idxw_v, idxp1_v, idxp2_v, wbuf, p1buf, p2buf, sem):
        wid = lax.axis_index("s") * NC + lax.axis_index("c")

        @pl.loop(0, n_chunks)
        def chunk(j):
            pltpu.sync_copy(widx.at[wid, pl.ds(j * KW, KW)], idxw_v)
            pltpu.sync_copy(p1idx.at[wid, pl.ds(j * KW, KW)], idxp1_v)
            pltpu.sync_copy(p2idx.at[wid, pl.ds(j * KW, KW)], idxp2_v)
            copies = []
            for t in range(KW):
                copies.append(pltpu.async_copy(
                    wtab.at[idxw_v.at[t]], wbuf.at[pl.ds(t * SL, SL)], sem))
                copies.append(pltpu.async_copy(
                    p1tab.at[idxp1_v.at[t]], p1buf.at[pl.ds(t * SL, SL)], sem))
                copies.append(pltpu.async_copy(
                    p2tab.at[idxp2_v.at[t]], p2buf.at[pl.ds(t * SL, SL)], sem))
            for cp in copies:
                cp.wait()
            base = wid * per_w + j * C
            pltpu.sync_copy(wbuf, out.at[pl.ds(base, C), pl.ds(0, word_dim)])
            pltpu.sync_copy(
                p1buf, out.at[pl.ds(base, C), pl.ds(word_dim, pos_dim)])
            pltpu.sync_copy(
                p2buf,
                out.at[pl.ds(base, C), pl.ds(word_dim + pos_dim, pos_dim)])

    return k


def kernel(word, pos1, pos2, word_table, pos1_table, pos2_table):
    b, s = word.shape
    word_dim = word_table.shape[1]
    pos_dim = pos1_table.shape[1]
    out_dim = word_dim + 2 * pos_dim
    n = b * s
    per_w = n // NW

    widx = word.reshape(NW, per_w // SL, SL)
    p1idx = pos1.reshape(NW, per_w // SL, SL)
    p2idx = pos2.reshape(NW, per_w // SL, SL)

    k = _make_kernel(n, word_dim, pos_dim, out_dim)
    out = k(widx, p1idx, p2idx, word_table, pos1_table, pos2_table)
    return out.reshape(b, s, out_dim)


# speedup vs baseline: 3.8037x; 3.8037x over previous
import functools

import jax
import jax.numpy as jnp
from jax import lax
from jax.experimental import pallas as pl
from jax.experimental.pallas import tpu as pltpu
from jax.experimental.pallas import tpu_sc as plsc

NC = 2
NS = 16
NW = NC * NS

SL = 128
KW = 4
C = KW * SL


def _make_kernel(n_rows, word_dim, pos_dim, out_dim):
    per_w = n_rows // NW
    n_chunks = per_w // C
    mesh = plsc.VectorSubcoreMesh(core_axis_name="c", subcore_axis_name="s",
                                  num_cores=NC, num_subcores=NS)

    @functools.partial(
        pl.kernel,
        out_type=jax.ShapeDtypeStruct((n_rows, out_dim), jnp.float32),
        mesh=mesh,
        compiler_params=pltpu.CompilerParams(use_tc_tiling_on_sc=False),
        scratch_types=[
            pltpu.VMEM((KW, SL), jnp.int32),
            pltpu.VMEM((KW, SL), jnp.int32),
            pltpu.VMEM((KW, SL), jnp.int32),
            pltpu.VMEM((C, word_dim), jnp.float32),
            pltpu.VMEM((C, pos_dim), jnp.float32),
            pltpu.VMEM((C, pos_dim), jnp.float32),
            pltpu.SemaphoreType.DMA,
        ],
    )
    def k(widx, p1idx, p2idx, wtab, p1tab, p2tab, out,
          idxw_v, idxp1_v, idxp2_v, wbuf, p1buf, p2buf, sem):
        wid = lax.axis_index("s") * NC + lax.axis_index("c")

        @pl.loop(0, n_chunks)
        def chunk(j):
            pltpu.sync_copy(widx.at[wid, pl.ds(j * KW, KW)], idxw_v)
            pltpu.sync_copy(p1idx.at[wid, pl.ds(j * KW, KW)], idxp1_v)
            pltpu.sync_copy(p2idx.at[wid, pl.ds(j * KW, KW)], idxp2_v)
            copies = []
            for t in range(KW):
                copies.append(pltpu.async_copy(
                    wtab.at[idxw_v.at[t]], wbuf.at[pl.ds(t * SL, SL)], sem))
                copies.append(pltpu.async_copy(
                    p1tab.at[idxp1_v.at[t]], p1buf.at[pl.ds(t * SL, SL)], sem))
                copies.append(pltpu.async_copy(
                    p2tab.at[idxp2_v.at[t]], p2buf.at[pl.ds(t * SL, SL)], sem))
            for cp in copies:
                cp.wait()
            base = wid * per_w + j * C
            pltpu.sync_copy(wbuf, out.at[pl.ds(base, C), pl.ds(0, word_dim)])
            pltpu.sync_copy(
                p1buf, out.at[pl.ds(base, C), pl.ds(word_dim, pos_dim)])
            pltpu.sync_copy(
                p2buf,
                out.at[pl.ds(base, C), pl.ds(word_dim + pos_dim, pos_dim)])

    return k


def kernel(word, pos1, pos2, word_table, pos1_table, pos2_table):
    b, s = word.shape
    word_dim = word_table.shape[1]
    pos_dim = pos1_table.shape[1]
    out_dim = word_dim + 2 * pos_dim
    n = b * s
    per_w = n // NW

    widx = word.reshape(NW, per_w // SL, SL)
    p1idx = pos1.reshape(NW, per_w // SL, SL)
    p2idx = pos2.reshape(NW, per_w // SL, SL)

    k = _make_kernel(n, word_dim, pos_dim, out_dim)
    out = k(widx, p1idx, p2idx, word_table, pos1_table, pos2_table)
    return out.reshape(b, s, out_dim)
